# baseline (speedup 1.0000x reference)
import jax
import jax.numpy as jnp
from jax.experimental import pallas as pl

B, N, M, D = 8, 2048, 2048, 3


def _chamfer_body(t_ref, s_ref, o1_ref, o2_ref):
    t = t_ref[0]
    s = s_ref[0]
    tn = jnp.sum(t * t, axis=1, keepdims=True)
    sn = jnp.sum(s * s, axis=0, keepdims=True)
    prod = jax.lax.dot_general(
        t, s, (((1,), (0,)), ((), ())),
        preferred_element_type=jnp.float32)
    d = tn + sn - 2.0 * prod
    d = jnp.maximum(d, 0.0)
    rowmin = jnp.min(d, axis=1)
    colmin = jnp.min(d, axis=0)
    s1 = jnp.sum(jnp.sqrt(rowmin))
    s2 = jnp.sum(jnp.sqrt(colmin))
    o1_ref[...] = jnp.full((1, 1, 128), s1, dtype=jnp.float32)
    o2_ref[...] = jnp.full((1, 1, 128), s2, dtype=jnp.float32)


def kernel(template, source):
    sT = jnp.swapaxes(source, 1, 2)
    o1, o2 = pl.pallas_call(
        _chamfer_body,
        grid=(B,),
        in_specs=[
            pl.BlockSpec((1, N, D), lambda b: (b, 0, 0)),
            pl.BlockSpec((1, D, M), lambda b: (b, 0, 0)),
        ],
        out_specs=[
            pl.BlockSpec((1, 1, 128), lambda b: (b, 0, 0)),
            pl.BlockSpec((1, 1, 128), lambda b: (b, 0, 0)),
        ],
        out_shape=[
            jax.ShapeDtypeStruct((B, 1, 128), jnp.float32),
            jax.ShapeDtypeStruct((B, 1, 128), jnp.float32),
        ],
    )(template, sT)
    cost_p0_p1 = jnp.sum(o1[:, 0, 0]) / (B * N)
    cost_p1_p0 = jnp.sum(o2[:, 0, 0]) / (B * M)
    return (cost_p0_p1 + cost_p1_p0) / 2.0


# speedup vs baseline: 1.1771x; 1.1771x over previous
import jax
import jax.numpy as jnp
from jax.experimental import pallas as pl

B, N, M, D = 8, 2048, 2048, 3


def _chamfer_body(t_ref, s_ref, o1_ref, o2_ref):
    t = t_ref[0]
    s = s_ref[0]
    tn = jnp.sum(t * t, axis=1, keepdims=True)
    sn = 0.25 * jnp.sum(s * s, axis=0, keepdims=True)
    prod = jax.lax.dot_general(
        t, s, (((1,), (0,)), ((), ())),
        preferred_element_type=jnp.float32)
    d = prod + tn + sn
    rowmin = jnp.maximum(jnp.min(d, axis=1), 0.0)
    colmin = jnp.maximum(jnp.min(d, axis=0), 0.0)
    s1 = jnp.sum(jnp.sqrt(rowmin))
    s2 = jnp.sum(jnp.sqrt(colmin))
    o1_ref[...] = jnp.full((1, 1, 128), s1, dtype=jnp.float32)
    o2_ref[...] = jnp.full((1, 1, 128), s2, dtype=jnp.float32)


def kernel(template, source):
    sT = jnp.swapaxes(source * -2.0, 1, 2)
    o1, o2 = pl.pallas_call(
        _chamfer_body,
        grid=(B,),
        in_specs=[
            pl.BlockSpec((1, N, D), lambda b: (b, 0, 0)),
            pl.BlockSpec((1, D, M), lambda b: (b, 0, 0)),
        ],
        out_specs=[
            pl.BlockSpec((1, 1, 128), lambda b: (b, 0, 0)),
            pl.BlockSpec((1, 1, 128), lambda b: (b, 0, 0)),
        ],
        out_shape=[
            jax.ShapeDtypeStruct((B, 1, 128), jnp.float32),
            jax.ShapeDtypeStruct((B, 1, 128), jnp.float32),
        ],
    )(template, sT)
    cost_p0_p1 = jnp.sum(o1[:, 0, 0]) / (B * N)
    cost_p1_p0 = jnp.sum(o2[:, 0, 0]) / (B * M)
    return (cost_p0_p1 + cost_p1_p0) / 2.0
